# baseline (device time: 46023 ns/iter reference)
import jax
import jax.numpy as jnp
from jax import lax
from jax.experimental import pallas as pl
from jax.experimental.pallas import tpu as pltpu

N_DEV = 4


def kernel(A, B):
    m, k = A.shape
    k2, n = B.shape

    def body(a_ref, b_ref, out_ref, comm_ref, send_sems, recv_sems):
        my_pos = lax.axis_index("i")
        left = lax.rem(my_pos - 1 + N_DEV, N_DEV)
        right = lax.rem(my_pos + 1, N_DEV)

        barrier_sem = pltpu.get_barrier_semaphore()
        for nbr in [left, right]:
            pl.semaphore_signal(
                barrier_sem, inc=1,
                device_id=(nbr,), device_id_type=pl.DeviceIdType.MESH,
            )
        pl.semaphore_wait(barrier_sem, 2)

        partial = jnp.dot(a_ref[:, :], b_ref[:, :],
                          preferred_element_type=jnp.float32)
        comm_ref[0, :, :] = partial
        out_ref[:, :] = partial

        for h in range(N_DEV - 1):
            rdma = pltpu.make_async_remote_copy(
                src_ref=comm_ref.at[h],
                dst_ref=comm_ref.at[h + 1],
                send_sem=send_sems.at[h],
                recv_sem=recv_sems.at[h],
                device_id=(right,),
                device_id_type=pl.DeviceIdType.MESH,
            )
            rdma.start()
            rdma.wait()
            out_ref[:, :] += comm_ref[h + 1, :, :]

        z = out_ref[:, :]
        out_ref[:, :] = z / (1.0 + jnp.exp(-z))

    return pl.pallas_call(
        body,
        out_shape=jax.ShapeDtypeStruct((m, n), jnp.float32),
        in_specs=[
            pl.BlockSpec(memory_space=pltpu.VMEM),
            pl.BlockSpec(memory_space=pltpu.VMEM),
        ],
        out_specs=pl.BlockSpec(memory_space=pltpu.VMEM),
        scratch_shapes=[
            pltpu.VMEM((N_DEV, m, n), jnp.float32),
            pltpu.SemaphoreType.DMA((N_DEV - 1,)),
            pltpu.SemaphoreType.DMA((N_DEV - 1,)),
        ],
        compiler_params=pltpu.CompilerParams(collective_id=0),
    )(A, B)


# device time: 21594 ns/iter; 2.1313x vs baseline; 2.1313x over previous
import jax
import jax.numpy as jnp
from jax import lax
from jax.experimental import pallas as pl
from jax.experimental.pallas import tpu as pltpu

N_DEV = 4


def kernel(A, B):
    m, k = A.shape
    _, n = B.shape
    mh = m // 2

    def body(a_ref, b_ref, out_ref, comm_ref, send_sems, recv_sems):
        my_pos = lax.axis_index("i")
        y_partner = my_pos ^ 1
        x_partner = 3 - my_pos

        barrier_sem = pltpu.get_barrier_semaphore()
        for nbr in [y_partner, x_partner]:
            pl.semaphore_signal(
                barrier_sem, inc=1,
                device_id=(nbr,), device_id_type=pl.DeviceIdType.MESH,
            )
        pl.semaphore_wait(barrier_sem, 2)

        out_ref[0:mh, :] = jnp.dot(a_ref[0:mh, :], b_ref[:, :],
                                   preferred_element_type=jnp.float32)

        p1 = pltpu.make_async_remote_copy(
            src_ref=out_ref.at[pl.ds(0, mh)],
            dst_ref=comm_ref.at[0, 0],
            send_sem=send_sems.at[0, 0],
            recv_sem=recv_sems.at[0, 0],
            device_id=(y_partner,),
            device_id_type=pl.DeviceIdType.MESH,
        )
        p1.start()

        out_ref[mh:m, :] = jnp.dot(a_ref[mh:m, :], b_ref[:, :],
                                   preferred_element_type=jnp.float32)

        q1 = pltpu.make_async_remote_copy(
            src_ref=out_ref.at[pl.ds(mh, mh)],
            dst_ref=comm_ref.at[0, 1],
            send_sem=send_sems.at[0, 1],
            recv_sem=recv_sems.at[0, 1],
            device_id=(x_partner,),
            device_id_type=pl.DeviceIdType.MESH,
        )
        q1.start()
        p1.wait()
        q1.wait()
        out_ref[0:mh, :] += comm_ref[0, 0, :, :]
        out_ref[mh:m, :] += comm_ref[0, 1, :, :]

        p2 = pltpu.make_async_remote_copy(
            src_ref=out_ref.at[pl.ds(0, mh)],
            dst_ref=comm_ref.at[1, 0],
            send_sem=send_sems.at[1, 0],
            recv_sem=recv_sems.at[1, 0],
            device_id=(x_partner,),
            device_id_type=pl.DeviceIdType.MESH,
        )
        q2 = pltpu.make_async_remote_copy(
            src_ref=out_ref.at[pl.ds(mh, mh)],
            dst_ref=comm_ref.at[1, 1],
            send_sem=send_sems.at[1, 1],
            recv_sem=recv_sems.at[1, 1],
            device_id=(y_partner,),
            device_id_type=pl.DeviceIdType.MESH,
        )
        p2.start()
        q2.start()
        p2.wait()
        q2.wait()
        out_ref[0:mh, :] += comm_ref[1, 0, :, :]
        out_ref[mh:m, :] += comm_ref[1, 1, :, :]

        z = out_ref[:, :]
        out_ref[:, :] = z / (1.0 + jnp.exp(-z))

    return pl.pallas_call(
        body,
        out_shape=jax.ShapeDtypeStruct((m, n), jnp.float32),
        in_specs=[
            pl.BlockSpec(memory_space=pltpu.VMEM),
            pl.BlockSpec(memory_space=pltpu.VMEM),
        ],
        out_specs=pl.BlockSpec(memory_space=pltpu.VMEM),
        scratch_shapes=[
            pltpu.VMEM((2, 2, mh, n), jnp.float32),
            pltpu.SemaphoreType.DMA((2, 2)),
            pltpu.SemaphoreType.DMA((2, 2)),
        ],
        compiler_params=pltpu.CompilerParams(collective_id=0),
    )(A, B)


# device time: 21235 ns/iter; 2.1673x vs baseline; 1.0169x over previous
import jax
import jax.numpy as jnp
from jax import lax
from jax.experimental import pallas as pl
from jax.experimental.pallas import tpu as pltpu

N_DEV = 4


def kernel(A, B):
    m, k = A.shape
    _, n = B.shape
    mh = m // 2

    def body(a_ref, b_ref, out_ref, comm_ref, send_sems, recv_sems):
        my_pos = lax.axis_index("i")
        y_partner = my_pos ^ 1
        x_partner = 3 - my_pos

        barrier_sem = pltpu.get_barrier_semaphore()
        for nbr in [y_partner, x_partner]:
            pl.semaphore_signal(
                barrier_sem, inc=1,
                device_id=(nbr,), device_id_type=pl.DeviceIdType.MESH,
            )
        pl.semaphore_wait(barrier_sem, 2)

        out_ref[0:mh, :] = jnp.dot(a_ref[0:mh, :], b_ref[:, :],
                                   preferred_element_type=jnp.float32)

        p1 = pltpu.make_async_remote_copy(
            src_ref=out_ref.at[pl.ds(0, mh)],
            dst_ref=comm_ref.at[0, 0],
            send_sem=send_sems.at[0, 0],
            recv_sem=recv_sems.at[0, 0],
            device_id=(y_partner,),
            device_id_type=pl.DeviceIdType.MESH,
        )
        p1.start()

        out_ref[mh:m, :] = jnp.dot(a_ref[mh:m, :], b_ref[:, :],
                                   preferred_element_type=jnp.float32)

        q1 = pltpu.make_async_remote_copy(
            src_ref=out_ref.at[pl.ds(mh, mh)],
            dst_ref=comm_ref.at[0, 1],
            send_sem=send_sems.at[0, 1],
            recv_sem=recv_sems.at[0, 1],
            device_id=(x_partner,),
            device_id_type=pl.DeviceIdType.MESH,
        )
        q1.start()

        p1.wait()
        out_ref[0:mh, :] += comm_ref[0, 0, :, :]
        p2 = pltpu.make_async_remote_copy(
            src_ref=out_ref.at[pl.ds(0, mh)],
            dst_ref=comm_ref.at[1, 0],
            send_sem=send_sems.at[1, 0],
            recv_sem=recv_sems.at[1, 0],
            device_id=(x_partner,),
            device_id_type=pl.DeviceIdType.MESH,
        )
        p2.start()

        q1.wait()
        out_ref[mh:m, :] += comm_ref[0, 1, :, :]
        q2 = pltpu.make_async_remote_copy(
            src_ref=out_ref.at[pl.ds(mh, mh)],
            dst_ref=comm_ref.at[1, 1],
            send_sem=send_sems.at[1, 1],
            recv_sem=recv_sems.at[1, 1],
            device_id=(y_partner,),
            device_id_type=pl.DeviceIdType.MESH,
        )
        q2.start()

        p2.wait()
        zp = out_ref[0:mh, :] + comm_ref[1, 0, :, :]
        out_ref[0:mh, :] = zp / (1.0 + jnp.exp(-zp))

        q2.wait()
        zq = out_ref[mh:m, :] + comm_ref[1, 1, :, :]
        out_ref[mh:m, :] = zq / (1.0 + jnp.exp(-zq))

    return pl.pallas_call(
        body,
        out_shape=jax.ShapeDtypeStruct((m, n), jnp.float32),
        in_specs=[
            pl.BlockSpec(memory_space=pltpu.VMEM),
            pl.BlockSpec(memory_space=pltpu.VMEM),
        ],
        out_specs=pl.BlockSpec(memory_space=pltpu.VMEM),
        scratch_shapes=[
            pltpu.VMEM((2, 2, mh, n), jnp.float32),
            pltpu.SemaphoreType.DMA((2, 2)),
            pltpu.SemaphoreType.DMA((2, 2)),
        ],
        compiler_params=pltpu.CompilerParams(collective_id=0),
    )(A, B)


# device time: 20121 ns/iter; 2.2873x vs baseline; 1.0554x over previous
import jax
import jax.numpy as jnp
from jax import lax
from jax.experimental import pallas as pl
from jax.experimental.pallas import tpu as pltpu

N_DEV = 4
NQ = 4


def kernel(A, B):
    m, k = A.shape
    _, n = B.shape
    mq = m // NQ

    def body(a_ref, b_ref, out_ref, comm_ref, send_sems, recv_sems):
        my_pos = lax.axis_index("i")
        y_partner = my_pos ^ 1
        x_partner = 3 - my_pos

        partner1 = [y_partner, y_partner, x_partner, x_partner]
        partner2 = [x_partner, x_partner, y_partner, y_partner]
        order = [0, 2, 1, 3]

        barrier_sem = pltpu.get_barrier_semaphore()
        for nbr in [y_partner, x_partner]:
            pl.semaphore_signal(
                barrier_sem, inc=1,
                device_id=(nbr,), device_id_type=pl.DeviceIdType.MESH,
            )
        pl.semaphore_wait(barrier_sem, 2)

        def rdma(q, stage, partner):
            return pltpu.make_async_remote_copy(
                src_ref=out_ref.at[pl.ds(q * mq, mq)],
                dst_ref=comm_ref.at[stage, q],
                send_sem=send_sems.at[stage, q],
                recv_sem=recv_sems.at[stage, q],
                device_id=(partner,),
                device_id_type=pl.DeviceIdType.MESH,
            )

        stage1 = []
        for q in range(NQ):
            out_ref[q * mq:(q + 1) * mq, :] = jnp.dot(
                a_ref[q * mq:(q + 1) * mq, :], b_ref[:, :],
                preferred_element_type=jnp.float32)
            s = rdma(q, 0, partner1[q])
            s.start()
            stage1.append(s)

        stage2 = [None] * NQ
        for q in order:
            stage1[q].wait_recv()
            stage1[q].wait_send()
            out_ref[q * mq:(q + 1) * mq, :] += comm_ref[0, q, :, :]
            s = rdma(q, 1, partner2[q])
            s.start()
            stage2[q] = s

        for q in order:
            stage2[q].wait_recv()
            stage2[q].wait_send()
            z = out_ref[q * mq:(q + 1) * mq, :] + comm_ref[1, q, :, :]
            out_ref[q * mq:(q + 1) * mq, :] = z / (1.0 + jnp.exp(-z))

    return pl.pallas_call(
        body,
        out_shape=jax.ShapeDtypeStruct((m, n), jnp.float32),
        in_specs=[
            pl.BlockSpec(memory_space=pltpu.VMEM),
            pl.BlockSpec(memory_space=pltpu.VMEM),
        ],
        out_specs=pl.BlockSpec(memory_space=pltpu.VMEM),
        scratch_shapes=[
            pltpu.VMEM((2, NQ, mq, n), jnp.float32),
            pltpu.SemaphoreType.DMA((2, NQ)),
            pltpu.SemaphoreType.DMA((2, NQ)),
        ],
        compiler_params=pltpu.CompilerParams(collective_id=0),
    )(A, B)
